# Initial kernel scaffold; baseline (speedup 1.0000x reference)
import jax
import jax.numpy as jnp
from jax.experimental import pallas as pl


def kernel(x, expert_idx, scale, expert_num):
    raise NotImplementedError("write your pallas kernel here")


# speedup vs baseline: 1.2672x; 1.2672x over previous
import functools

import jax
import jax.numpy as jnp
from jax import lax
from jax.experimental import pallas as pl
from jax.experimental.pallas import tpu as pltpu
from jax.experimental.pallas import tpu_sc as plsc

NR = 8192
H = 4096
K = 2
N = NR * K
E = 16
NC = 2
NS = 16
NW = NC * NS
CH = N // NW
G = 8

_mesh = plsc.VectorSubcoreMesh(core_axis_name="c", subcore_axis_name="s")
_params = pltpu.CompilerParams(needs_layout_passes=False)


def _wid():
    return lax.axis_index("c") * NS + lax.axis_index("s")


@functools.partial(
    pl.kernel,
    out_type=jax.ShapeDtypeStruct((NW * E,), jnp.int32),
    mesh=_mesh,
    compiler_params=_params,
    scratch_types=[
        pltpu.VMEM((CH,), jnp.int32),
        pltpu.VMEM((E,), jnp.int32),
    ],
)
def _hist(eidx_hbm, hist_hbm, ev_ref, out_ref):
    wid = _wid()
    pltpu.sync_copy(eidx_hbm.at[pl.ds(wid * CH, CH)], ev_ref)
    lanes = lax.iota(jnp.int32, 16)

    def body(j, accs):
        ev = ev_ref[pl.ds(j * 16, 16)]
        return tuple(accs[e] + (ev == e).astype(jnp.int32) for e in range(E))

    accs = lax.fori_loop(
        0, CH // 16, body, tuple(jnp.zeros((16,), jnp.int32) for _ in range(E)))
    cnt = jnp.zeros((16,), jnp.int32)
    for e in range(E):
        cnt = jnp.where(lanes == e, jnp.sum(accs[e]), cnt)
    out_ref[...] = cnt
    pltpu.sync_copy(out_ref, hist_hbm.at[pl.ds(wid * E, E)])


@functools.partial(
    pl.kernel,
    out_type=(
        jax.ShapeDtypeStruct((N,), jnp.int32),
        jax.ShapeDtypeStruct((N,), jnp.int32),
        jax.ShapeDtypeStruct((E,), jnp.int32),
    ),
    mesh=_mesh,
    compiler_params=_params,
    scratch_types=[
        pltpu.VMEM((CH,), jnp.int32),
        pltpu.VMEM((NW * E,), jnp.int32),
        pltpu.VMEM((4, 128), jnp.int32),
        pltpu.VMEM((CH,), jnp.int32),
        pltpu.VMEM((E,), jnp.int32),
        pltpu.SemaphoreType.DMA,
    ],
)
def _route(eidx_hbm, hist_hbm, eri_hbm, rmap_hbm, cnt_hbm,
           ev_ref, hist_ref, pos_ref, src_ref, tmp_ref, sem):
    wid = _wid()
    base = wid * CH
    lanes = lax.iota(jnp.int32, 16)

    pltpu.sync_copy(eidx_hbm.at[pl.ds(base, CH)], ev_ref)
    pltpu.sync_copy(hist_hbm, hist_ref)

    prefix = jnp.zeros((16,), jnp.int32)
    totals = jnp.zeros((16,), jnp.int32)
    for t in range(NW):
        h = hist_ref[pl.ds(t * E, E)]
        totals = totals + h
        prefix = prefix + jnp.where(t < wid, h, jnp.zeros_like(h))
    base_v = plsc.cumsum(totals) - totals + prefix

    @pl.when(wid == 0)
    def _():
        tmp_ref[...] = totals
        pltpu.sync_copy(tmp_ref, cnt_hbm)

    def fill(j, _):
        gl = j * 16 + lanes
        src_ref[pl.ds(j * 16, 16)] = (base + gl) // 2
        return 0

    lax.fori_loop(0, CH // 16, fill, 0)

    for e in range(E):
        b0 = jnp.sum(jnp.where(lanes == e, base_v, jnp.zeros_like(base_v)))

        def body(j, b, e=e):
            ev = ev_ref[pl.ds(j * 16, 16)]
            m = ev == e
            mi = m.astype(jnp.int32)
            c = plsc.cumsum(mi)
            pv = b + c - 1
            gl = j * 16 + lanes
            plsc.store_scatter(pos_ref, [gl // 128, gl % 128], pv, mask=m)
            return b + jnp.sum(mi)

        lax.fori_loop(0, CH // 16, body, b0)

    for r in range(4):
        pltpu.sync_copy(pos_ref.at[r], eri_hbm.at[pl.ds(base + r * 128, 128)])
    for r in range(4):
        pltpu.async_copy(src_ref.at[pl.ds(r * 128, 128)],
                         rmap_hbm.at[pos_ref.at[r]], sem).wait()


@functools.partial(
    pl.kernel,
    out_type=(
        jax.ShapeDtypeStruct((N, H), jnp.float32),
        jax.ShapeDtypeStruct((N,), jnp.float32),
    ),
    mesh=_mesh,
    compiler_params=_params,
    scratch_types=[
        pltpu.VMEM((CH,), jnp.int32),
        pltpu.VMEM((NR,), jnp.float32),
        pltpu.VMEM((CH,), jnp.float32),
        pltpu.VMEM((G, H), jnp.float32),
        pltpu.SemaphoreType.DMA,
    ],
)
def _gather(x_hbm, scale_hbm, rmap_hbm, ex_hbm, esc_hbm,
            rmap_ref, scl_ref, esc_ref, rows_ref, sem):
    wid = _wid()
    base = wid * CH

    pltpu.sync_copy(rmap_hbm.at[pl.ds(base, CH)], rmap_ref)
    pltpu.sync_copy(scale_hbm, scl_ref)

    def fill(j, _):
        rv = rmap_ref[pl.ds(j * 16, 16)]
        esc_ref[pl.ds(j * 16, 16)] = plsc.load_gather(scl_ref, [rv])
        return 0

    lax.fori_loop(0, CH // 16, fill, 0)
    pltpu.sync_copy(esc_ref, esc_hbm.at[pl.ds(base, CH)])

    def step(it, _):
        pltpu.async_copy(x_hbm.at[rmap_ref.at[pl.ds(it * G, G)]],
                         rows_ref, sem).wait()
        pltpu.sync_copy(rows_ref, ex_hbm.at[pl.ds(base + it * G, G)])
        return 0

    lax.fori_loop(0, CH // G, step, 0)


def kernel(x, expert_idx, scale, expert_num):
    eidx = expert_idx.reshape(-1).astype(jnp.int32)
    hist = _hist(eidx)
    eri, rmap, cnt = _route(eidx, hist)
    ex, esc = _gather(x, scale, rmap)
    etn = jnp.where(jnp.arange(E) < expert_num, cnt, 0).astype(jnp.int64)
    return ex, eri, etn, esc


# speedup vs baseline: 1.4868x; 1.1732x over previous
import functools

import jax
import jax.numpy as jnp
from jax import lax
from jax.experimental import pallas as pl
from jax.experimental.pallas import tpu as pltpu
from jax.experimental.pallas import tpu_sc as plsc

NR = 8192
H = 4096
K = 2
N = NR * K
E = 16
NC = 2
NS = 16
NW = NC * NS
CH = N // NW
G = 8

_mesh = plsc.VectorSubcoreMesh(core_axis_name="c", subcore_axis_name="s")
_params = pltpu.CompilerParams(needs_layout_passes=False)


def _wid():
    return lax.axis_index("c") * NS + lax.axis_index("s")


@functools.partial(
    pl.kernel,
    out_type=jax.ShapeDtypeStruct((NW * E,), jnp.int32),
    mesh=_mesh,
    compiler_params=_params,
    scratch_types=[
        pltpu.VMEM((CH,), jnp.int32),
        pltpu.VMEM((E,), jnp.int32),
    ],
)
def _hist(eidx_hbm, hist_hbm, ev_ref, out_ref):
    wid = _wid()
    pltpu.sync_copy(eidx_hbm.at[pl.ds(wid * CH, CH)], ev_ref)
    lanes = lax.iota(jnp.int32, 16)

    def body(j, accs):
        ev = ev_ref[pl.ds(j * 16, 16)]
        return tuple(accs[e] + (ev == e).astype(jnp.int32) for e in range(E))

    accs = lax.fori_loop(
        0, CH // 16, body, tuple(jnp.zeros((16,), jnp.int32) for _ in range(E)))
    cnt = jnp.zeros((16,), jnp.int32)
    for e in range(E):
        cnt = jnp.where(lanes == e, jnp.sum(accs[e]), cnt)
    out_ref[...] = cnt
    pltpu.sync_copy(out_ref, hist_hbm.at[pl.ds(wid * E, E)])


@functools.partial(
    pl.kernel,
    out_type=(
        jax.ShapeDtypeStruct((N,), jnp.int32),
        jax.ShapeDtypeStruct((N,), jnp.int32),
        jax.ShapeDtypeStruct((E,), jnp.int32),
    ),
    mesh=_mesh,
    compiler_params=_params,
    scratch_types=[
        pltpu.VMEM((CH,), jnp.int32),
        pltpu.VMEM((NW * E,), jnp.int32),
        pltpu.VMEM((4, 128), jnp.int32),
        pltpu.VMEM((CH,), jnp.int32),
        pltpu.VMEM((E,), jnp.int32),
        pltpu.SemaphoreType.DMA,
    ],
)
def _route(eidx_hbm, hist_hbm, eri_hbm, rmap_hbm, cnt_hbm,
           ev_ref, hist_ref, pos_ref, src_ref, tmp_ref, sem):
    wid = _wid()
    base = wid * CH
    lanes = lax.iota(jnp.int32, 16)

    pltpu.sync_copy(eidx_hbm.at[pl.ds(base, CH)], ev_ref)
    pltpu.sync_copy(hist_hbm, hist_ref)

    prefix = jnp.zeros((16,), jnp.int32)
    totals = jnp.zeros((16,), jnp.int32)
    for t in range(NW):
        h = hist_ref[pl.ds(t * E, E)]
        totals = totals + h
        prefix = prefix + jnp.where(t < wid, h, jnp.zeros_like(h))
    base_v = plsc.cumsum(totals) - totals + prefix

    @pl.when(wid == 0)
    def _():
        tmp_ref[...] = totals
        pltpu.sync_copy(tmp_ref, cnt_hbm)

    def fill(j, _):
        gl = j * 16 + lanes
        src_ref[pl.ds(j * 16, 16)] = (base + gl) // 2
        return 0

    lax.fori_loop(0, CH // 16, fill, 0)

    for e in range(E):
        b0 = jnp.sum(jnp.where(lanes == e, base_v, jnp.zeros_like(base_v)))

        def body(j, b, e=e):
            ev = ev_ref[pl.ds(j * 16, 16)]
            m = ev == e
            mi = m.astype(jnp.int32)
            c = plsc.cumsum(mi)
            pv = b + c - 1
            gl = j * 16 + lanes
            plsc.store_scatter(pos_ref, [gl // 128, gl % 128], pv, mask=m)
            return b + jnp.sum(mi)

        lax.fori_loop(0, CH // 16, body, b0)

    for r in range(4):
        pltpu.sync_copy(pos_ref.at[r], eri_hbm.at[pl.ds(base + r * 128, 128)])
    for r in range(4):
        pltpu.async_copy(src_ref.at[pl.ds(r * 128, 128)],
                         rmap_hbm.at[pos_ref.at[r]], sem).wait()


@functools.partial(
    pl.kernel,
    out_type=(
        jax.ShapeDtypeStruct((N, H), jnp.float32),
        jax.ShapeDtypeStruct((N,), jnp.float32),
    ),
    mesh=_mesh,
    compiler_params=_params,
    scratch_types=[
        pltpu.VMEM((CH,), jnp.int32),
        pltpu.VMEM((NR,), jnp.float32),
        pltpu.VMEM((CH,), jnp.float32),
        pltpu.VMEM((G, H), jnp.float32),
        pltpu.VMEM((G, H), jnp.float32),
        pltpu.SemaphoreType.DMA,
        pltpu.SemaphoreType.DMA,
    ],
)
def _gather(x_hbm, scale_hbm, rmap_hbm, ex_hbm, esc_hbm,
            rmap_ref, scl_ref, esc_ref, rows0_ref, rows1_ref,
            sem_g, sem_w):
    wid = _wid()
    base = wid * CH

    pltpu.sync_copy(rmap_hbm.at[pl.ds(base, CH)], rmap_ref)
    pltpu.sync_copy(scale_hbm, scl_ref)

    def fill(j, _):
        rv = rmap_ref[pl.ds(j * 16, 16)]
        esc_ref[pl.ds(j * 16, 16)] = plsc.load_gather(scl_ref, [rv])
        return 0

    lax.fori_loop(0, CH // 16, fill, 0)
    pltpu.sync_copy(esc_ref, esc_hbm.at[pl.ds(base, CH)])

    bufs = (rows0_ref, rows1_ref)
    nit = CH // G

    def gat(i, buf):
        return pltpu.async_copy(x_hbm.at[rmap_ref.at[pl.ds(i * G, G)]],
                                buf, sem_g)

    def wrt(i, buf):
        return pltpu.async_copy(buf, ex_hbm.at[pl.ds(base + i * G, G)], sem_w)

    def wrt_wait(i, buf):
        pltpu.make_async_copy(buf, ex_hbm.at[pl.ds(base + i * G, G)],
                              sem_w).wait()

    gat(0, bufs[0])
    gat(1, bufs[1])

    @pl.loop(0, nit, step=2)
    def _(i0):
        for b in range(2):
            i = i0 + b
            buf = bufs[b]
            pltpu.make_async_copy(x_hbm.at[rmap_ref.at[pl.ds(i * G, G)]],
                                  buf, sem_g).wait()
            wrt(i, buf)

            @pl.when(i + 2 < nit)
            def _():
                wrt_wait(i, buf)
                gat(i + 2, buf)

    wrt_wait(nit - 2, bufs[0])
    wrt_wait(nit - 1, bufs[1])


def kernel(x, expert_idx, scale, expert_num):
    eidx = expert_idx.reshape(-1).astype(jnp.int32)
    hist = _hist(eidx)
    eri, rmap, cnt = _route(eidx, hist)
    ex, esc = _gather(x, scale, rmap)
    etn = jnp.where(jnp.arange(E) < expert_num, cnt, 0).astype(jnp.int64)
    return ex, eri, etn, esc


# speedup vs baseline: 1.4987x; 1.0080x over previous
import functools

import jax
import jax.numpy as jnp
from jax import lax
from jax.experimental import pallas as pl
from jax.experimental.pallas import tpu as pltpu
from jax.experimental.pallas import tpu_sc as plsc

NR = 8192
H = 4096
K = 2
N = NR * K
E = 16
NC = 2
NS = 16
NW = NC * NS
CH = N // NW
G = 8

_mesh = plsc.VectorSubcoreMesh(core_axis_name="c", subcore_axis_name="s")
_params = pltpu.CompilerParams(needs_layout_passes=False)


def _wid():
    return lax.axis_index("c") * NS + lax.axis_index("s")


@functools.partial(
    pl.kernel,
    out_type=jax.ShapeDtypeStruct((NW * E,), jnp.int32),
    mesh=_mesh,
    compiler_params=_params,
    scratch_types=[
        pltpu.VMEM((CH,), jnp.int32),
        pltpu.VMEM((E,), jnp.int32),
    ],
)
def _hist(eidx_hbm, hist_hbm, ev_ref, out_ref):
    wid = _wid()
    pltpu.sync_copy(eidx_hbm.at[pl.ds(wid * CH, CH)], ev_ref)
    lanes = lax.iota(jnp.int32, 16)

    def body(j, accs):
        ev = ev_ref[pl.ds(j * 16, 16)]
        return tuple(accs[e] + (ev == e).astype(jnp.int32) for e in range(E))

    accs = lax.fori_loop(
        0, CH // 16, body, tuple(jnp.zeros((16,), jnp.int32) for _ in range(E)))
    cnt = jnp.zeros((16,), jnp.int32)
    for e in range(E):
        cnt = jnp.where(lanes == e, jnp.sum(accs[e]), cnt)
    out_ref[...] = cnt
    pltpu.sync_copy(out_ref, hist_hbm.at[pl.ds(wid * E, E)])


@functools.partial(
    pl.kernel,
    out_type=(
        jax.ShapeDtypeStruct((N,), jnp.int32),
        jax.ShapeDtypeStruct((N,), jnp.int32),
        jax.ShapeDtypeStruct((E,), jnp.int32),
    ),
    mesh=_mesh,
    compiler_params=_params,
    scratch_types=[
        pltpu.VMEM((CH,), jnp.int32),
        pltpu.VMEM((NW * E,), jnp.int32),
        pltpu.VMEM((4, 128), jnp.int32),
        pltpu.VMEM((CH,), jnp.int32),
        pltpu.VMEM((E,), jnp.int32),
        pltpu.SemaphoreType.DMA,
    ],
)
def _route(eidx_hbm, hist_hbm, eri_hbm, rmap_hbm, cnt_hbm,
           ev_ref, hist_ref, pos_ref, src_ref, tmp_ref, sem):
    wid = _wid()
    base = wid * CH
    lanes = lax.iota(jnp.int32, 16)

    pltpu.sync_copy(eidx_hbm.at[pl.ds(base, CH)], ev_ref)
    pltpu.sync_copy(hist_hbm, hist_ref)

    prefix = jnp.zeros((16,), jnp.int32)
    totals = jnp.zeros((16,), jnp.int32)
    for t in range(NW):
        h = hist_ref[pl.ds(t * E, E)]
        totals = totals + h
        prefix = prefix + jnp.where(t < wid, h, jnp.zeros_like(h))
    base_v = plsc.cumsum(totals) - totals + prefix

    @pl.when(wid == 0)
    def _():
        tmp_ref[...] = totals
        pltpu.sync_copy(tmp_ref, cnt_hbm)

    def fill(j, _):
        gl = j * 16 + lanes
        src_ref[pl.ds(j * 16, 16)] = (base + gl) // 2
        return 0

    lax.fori_loop(0, CH // 16, fill, 0)

    for e in range(E):
        b0 = jnp.sum(jnp.where(lanes == e, base_v, jnp.zeros_like(base_v)))

        def body(j, b, e=e):
            ev = ev_ref[pl.ds(j * 16, 16)]
            m = ev == e
            c = plsc.cumsum(m.astype(jnp.int32))
            pv = b + c - 1
            gl = j * 16 + lanes
            plsc.store_scatter(pos_ref, [gl // 128, gl % 128], pv, mask=m)
            return b + plsc.all_reduce_population_count(m)

        lax.fori_loop(0, CH // 16, body, jnp.broadcast_to(b0, (16,)))

    for r in range(4):
        pltpu.sync_copy(pos_ref.at[r], eri_hbm.at[pl.ds(base + r * 128, 128)])
    for r in range(4):
        pltpu.async_copy(src_ref.at[pl.ds(r * 128, 128)],
                         rmap_hbm.at[pos_ref.at[r]], sem).wait()


@functools.partial(
    pl.kernel,
    out_type=(
        jax.ShapeDtypeStruct((N, H), jnp.float32),
        jax.ShapeDtypeStruct((N,), jnp.float32),
    ),
    mesh=_mesh,
    compiler_params=_params,
    scratch_types=[
        pltpu.VMEM((CH,), jnp.int32),
        pltpu.VMEM((NR,), jnp.float32),
        pltpu.VMEM((CH,), jnp.float32),
        pltpu.VMEM((G, H), jnp.float32),
        pltpu.VMEM((G, H), jnp.float32),
        pltpu.VMEM((G, H), jnp.float32),
        pltpu.SemaphoreType.DMA,
        pltpu.SemaphoreType.DMA,
    ],
)
def _gather(x_hbm, scale_hbm, rmap_hbm, ex_hbm, esc_hbm,
            rmap_ref, scl_ref, esc_ref, rows0_ref, rows1_ref, rows2_ref,
            sem_g, sem_w):
    wid = _wid()
    base = wid * CH

    pltpu.sync_copy(rmap_hbm.at[pl.ds(base, CH)], rmap_ref)
    pltpu.sync_copy(scale_hbm, scl_ref)

    def fill(j, _):
        rv = rmap_ref[pl.ds(j * 16, 16)]
        esc_ref[pl.ds(j * 16, 16)] = plsc.load_gather(scl_ref, [rv])
        return 0

    lax.fori_loop(0, CH // 16, fill, 0)
    pltpu.sync_copy(esc_ref, esc_hbm.at[pl.ds(base, CH)])

    bufs = (rows0_ref, rows1_ref, rows2_ref)
    nb = len(bufs)
    nit = CH // G

    def gat(i, buf):
        return pltpu.async_copy(x_hbm.at[rmap_ref.at[pl.ds(i * G, G)]],
                                buf, sem_g)

    def wrt(i, buf):
        return pltpu.async_copy(buf, ex_hbm.at[pl.ds(base + i * G, G)], sem_w)

    def wrt_wait(i, buf):
        pltpu.make_async_copy(buf, ex_hbm.at[pl.ds(base + i * G, G)],
                              sem_w).wait()

    def gat_wait(i, buf):
        pltpu.make_async_copy(x_hbm.at[rmap_ref.at[pl.ds(i * G, G)]],
                              buf, sem_g).wait()

    main = (nit // nb) * nb
    for b in range(nb):
        gat(b, bufs[b])

    @pl.loop(0, main, step=nb)
    def _(i0):
        for b in range(nb):
            i = i0 + b
            buf = bufs[b]
            gat_wait(i, buf)
            wrt(i, buf)

            @pl.when(i + nb < nit)
            def _():
                wrt_wait(i, buf)
                gat(i + nb, buf)

    for i in range(main, nit):
        gat_wait(i, bufs[i % nb])
        wrt(i, bufs[i % nb])
    for i in range(nit - nb, nit):
        wrt_wait(i, bufs[i % nb])


def kernel(x, expert_idx, scale, expert_num):
    eidx = expert_idx.reshape(-1).astype(jnp.int32)
    hist = _hist(eidx)
    eri, rmap, cnt = _route(eidx, hist)
    ex, esc = _gather(x, scale, rmap)
    etn = jnp.where(jnp.arange(E) < expert_num, cnt, 0).astype(jnp.int64)
    return ex, eri, etn, esc


# speedup vs baseline: 1.7696x; 1.1808x over previous
import functools

import jax
import jax.numpy as jnp
from jax import lax
from jax.experimental import pallas as pl
from jax.experimental.pallas import tpu as pltpu
from jax.experimental.pallas import tpu_sc as plsc

NR = 8192
H = 4096
K = 2
N = NR * K
E = 16
NC = 2
NS = 16
NW = NC * NS
CH = N // NW
TOK = CH // K
G = 8

_mesh = plsc.VectorSubcoreMesh(core_axis_name="c", subcore_axis_name="s")
_params = pltpu.CompilerParams(needs_layout_passes=False)


def _wid():
    return lax.axis_index("c") * NS + lax.axis_index("s")


@functools.partial(
    pl.kernel,
    out_type=jax.ShapeDtypeStruct((NW * E,), jnp.int32),
    mesh=_mesh,
    compiler_params=_params,
    scratch_types=[
        pltpu.VMEM((CH,), jnp.int32),
        pltpu.VMEM((E,), jnp.int32),
    ],
)
def _hist(eidx_hbm, hist_hbm, ev_ref, out_ref):
    wid = _wid()
    pltpu.sync_copy(eidx_hbm.at[pl.ds(wid * CH, CH)], ev_ref)
    lanes = lax.iota(jnp.int32, 16)

    def body(j, accs):
        ev = ev_ref[pl.ds(j * 16, 16)]
        return tuple(accs[e] + (ev == e).astype(jnp.int32) for e in range(E))

    accs = lax.fori_loop(
        0, CH // 16, body, tuple(jnp.zeros((16,), jnp.int32) for _ in range(E)))
    cnt = jnp.zeros((16,), jnp.int32)
    for e in range(E):
        cnt = jnp.where(lanes == e, jnp.sum(accs[e]), cnt)
    out_ref[...] = cnt
    pltpu.sync_copy(out_ref, hist_hbm.at[pl.ds(wid * E, E)])


@functools.partial(
    pl.kernel,
    out_type=(
        jax.ShapeDtypeStruct((N,), jnp.int32),
        jax.ShapeDtypeStruct((E,), jnp.int32),
    ),
    mesh=_mesh,
    compiler_params=_params,
    scratch_types=[
        pltpu.VMEM((CH,), jnp.int32),
        pltpu.VMEM((NW * E,), jnp.int32),
        pltpu.VMEM((CH,), jnp.int32),
        pltpu.VMEM((E,), jnp.int32),
    ],
)
def _route(eidx_hbm, hist_hbm, eri_hbm, cnt_hbm,
           ev_ref, hist_ref, pos_ref, tmp_ref):
    wid = _wid()
    base = wid * CH
    lanes = lax.iota(jnp.int32, 16)

    pltpu.sync_copy(eidx_hbm.at[pl.ds(base, CH)], ev_ref)
    pltpu.sync_copy(hist_hbm, hist_ref)

    prefix = jnp.zeros((16,), jnp.int32)
    totals = jnp.zeros((16,), jnp.int32)
    for t in range(NW):
        h = hist_ref[pl.ds(t * E, E)]
        totals = totals + h
        prefix = prefix + jnp.where(t < wid, h, jnp.zeros_like(h))
    base_v = plsc.cumsum(totals) - totals + prefix

    @pl.when(wid == 0)
    def _():
        tmp_ref[...] = totals
        pltpu.sync_copy(tmp_ref, cnt_hbm)

    bases0 = tuple(
        jnp.broadcast_to(
            jnp.sum(jnp.where(lanes == e, base_v, jnp.zeros_like(base_v))),
            (16,))
        for e in range(E))

    def body(j, bases):
        ev = ev_ref[pl.ds(j * 16, 16)]
        pos_v = jnp.zeros((16,), jnp.int32)
        nxt = []
        for e in range(E):
            m = ev == e
            c = plsc.cumsum(m.astype(jnp.int32))
            pos_v = jnp.where(m, bases[e] + c - 1, pos_v)
            nxt.append(bases[e] + plsc.all_reduce_population_count(m))
        pos_ref[pl.ds(j * 16, 16)] = pos_v
        return tuple(nxt)

    lax.fori_loop(0, CH // 16, body, bases0)
    pltpu.sync_copy(pos_ref, eri_hbm.at[pl.ds(base, CH)])


@functools.partial(
    pl.kernel,
    out_type=(
        jax.ShapeDtypeStruct((N, H), jnp.float32),
        jax.ShapeDtypeStruct((N,), jnp.float32),
    ),
    mesh=_mesh,
    compiler_params=_params,
    scratch_types=[
        pltpu.VMEM((CH,), jnp.int32),
        pltpu.VMEM((TOK,), jnp.float32),
        pltpu.VMEM((TOK // G, G), jnp.int32),
        pltpu.VMEM((TOK // G, G), jnp.int32),
        pltpu.VMEM((TOK // 128, 128), jnp.int32),
        pltpu.VMEM((TOK // 128, 128), jnp.int32),
        pltpu.VMEM((G, H), jnp.float32),
        pltpu.VMEM((G, H), jnp.float32),
        pltpu.VMEM((G, H), jnp.float32),
        pltpu.SemaphoreType.DMA,
        pltpu.SemaphoreType.DMA,
        pltpu.SemaphoreType.DMA,
    ],
)
def _gather(x_hbm, scale_hbm, eri_hbm, ex_hbm, esc_hbm,
            pos_ref, scl_ref, ide_ref, ido_ref, ide128_ref, ido128_ref,
            rows0_ref, rows1_ref, rows2_ref, sem_g, sem_w, sem_s):
    wid = _wid()
    base = wid * CH
    tok0 = wid * TOK
    lanes = lax.iota(jnp.int32, 16)

    pltpu.sync_copy(eri_hbm.at[pl.ds(base, CH)], pos_ref)
    pltpu.sync_copy(scale_hbm.at[pl.ds(tok0, TOK)], scl_ref)

    def fill(j, _):
        pv = pos_ref[pl.ds(j * 16, 16)]
        l = j * 16 + lanes
        tok = l // 2
        m_e = (l % 2) == 0
        m_o = jnp.logical_not(m_e)
        plsc.store_scatter(ide_ref, [tok // G, tok % G], pv, mask=m_e)
        plsc.store_scatter(ido_ref, [tok // G, tok % G], pv, mask=m_o)
        plsc.store_scatter(ide128_ref, [tok // 128, tok % 128], pv, mask=m_e)
        plsc.store_scatter(ido128_ref, [tok // 128, tok % 128], pv, mask=m_o)
        return 0

    lax.fori_loop(0, CH // 16, fill, 0)

    for r in range(TOK // 128):
        src = scl_ref.at[pl.ds(r * 128, 128)]
        pltpu.async_copy(src, esc_hbm.at[ide128_ref.at[r]], sem_s)
        pltpu.async_copy(src, esc_hbm.at[ido128_ref.at[r]], sem_s)

    bufs = (rows0_ref, rows1_ref, rows2_ref)
    nb = len(bufs)
    nit = TOK // G

    def gat(i, buf):
        pltpu.async_copy(x_hbm.at[pl.ds(tok0 + i * G, G)], buf, sem_g)

    def gat_wait(i, buf):
        pltpu.make_async_copy(x_hbm.at[pl.ds(tok0 + i * G, G)], buf,
                              sem_g).wait()

    def wrt(i, buf):
        pltpu.async_copy(buf, ex_hbm.at[ide_ref.at[i]], sem_w)
        pltpu.async_copy(buf, ex_hbm.at[ido_ref.at[i]], sem_w)

    def wrt_wait(i, buf):
        pltpu.make_async_copy(buf, ex_hbm.at[ide_ref.at[i]], sem_w).wait()
        pltpu.make_async_copy(buf, ex_hbm.at[ido_ref.at[i]], sem_w).wait()

    main = (nit // nb) * nb
    for b in range(nb):
        gat(b, bufs[b])

    @pl.loop(0, main, step=nb)
    def _(i0):
        for b in range(nb):
            i = i0 + b
            buf = bufs[b]
            gat_wait(i, buf)
            wrt(i, buf)

            @pl.when(i + nb < nit)
            def _():
                wrt_wait(i, buf)
                gat(i + nb, buf)

    for i in range(main, nit):
        gat_wait(i, bufs[i % nb])
        wrt(i, bufs[i % nb])
    for i in range(nit - nb, nit):
        wrt_wait(i, bufs[i % nb])
    for r in range(TOK // 128):
        src = scl_ref.at[pl.ds(r * 128, 128)]
        pltpu.make_async_copy(src, esc_hbm.at[ide128_ref.at[r]], sem_s).wait()
        pltpu.make_async_copy(src, esc_hbm.at[ido128_ref.at[r]], sem_s).wait()


def kernel(x, expert_idx, scale, expert_num):
    eidx = expert_idx.reshape(-1).astype(jnp.int32)
    hist = _hist(eidx)
    eri, cnt = _route(eidx, hist)
    ex, esc = _gather(x, scale, eri)
    etn = jnp.where(jnp.arange(E) < expert_num, cnt, 0).astype(jnp.int64)
    return ex, eri, etn, esc
